# baseline (speedup 1.0000x reference)
import functools

import jax
import jax.numpy as jnp
import numpy as np
from jax.experimental import pallas as pl


def _fc1_body(h_ref, w_ref, b_ref, o_ref):
    i = pl.program_id(0)

    @pl.when(i == 0)
    def _init():
        o_ref[...] = jnp.broadcast_to(b_ref[...], o_ref.shape)

    o_ref[...] += jnp.dot(h_ref[...], w_ref[...],
                          preferred_element_type=jnp.float32)


def _fc1(h, w, b):
    B, M = h.shape
    C = w.shape[1]
    CHUNK = 5120
    grid = M // CHUNK
    return pl.pallas_call(
        _fc1_body,
        grid=(grid,),
        in_specs=[
            pl.BlockSpec((B, CHUNK), lambda i: (0, i)),
            pl.BlockSpec((CHUNK, C), lambda i: (i, 0)),
            pl.BlockSpec((C,), lambda i: (0,)),
        ],
        out_specs=pl.BlockSpec((B, C), lambda i: (0, 0)),
        out_shape=jax.ShapeDtypeStruct((B, C), jnp.float32),
    )(h, w, b)


def _cheb(h, src, dst, norm, W, b, n):
    def prop(t):
        msg = t[src] * norm[:, None, None]
        return jax.ops.segment_sum(msg, dst, num_segments=n)
    Tx0 = h
    out = jnp.einsum('nbf,fg->nbg', Tx0, W[0])
    Tx1 = prop(Tx0)
    out = out + jnp.einsum('nbf,fg->nbg', Tx1, W[1])
    for k in range(2, W.shape[0]):
        Tx2 = 2.0 * prop(Tx1) - Tx0
        out = out + jnp.einsum('nbf,fg->nbg', Tx2, W[k])
        Tx0, Tx1 = Tx1, Tx2
    return out + b


def kernel(x, edge_index, W1, b1, W2, b2, fc1_w, fc1_b, fc2_w, fc2_b):
    n = x.shape[1]
    t = x.shape[2]
    src = edge_index[0]
    dst = edge_index[1]
    deg = jax.ops.segment_sum(jnp.ones((src.shape[0],), jnp.float32), dst,
                              num_segments=n)
    dinv = jnp.where(deg > 0, 1.0 / jnp.sqrt(deg), 0.0)
    norm = -dinv[src] * dinv[dst]
    cosmat = jnp.cos(2.0 * np.pi * jnp.outer(jnp.arange(t), jnp.arange(t))
                     / t).astype(jnp.float32)
    xr = jnp.einsum('bnt,tf->bnf', x, cosmat)
    h = jnp.transpose(xr, (1, 0, 2))
    h = jax.nn.relu(_cheb(h, src, dst, norm, W1, b1, n))
    h = jax.nn.relu(_cheb(h, src, dst, norm, W2, b2, n))
    h = jnp.transpose(h, (1, 0, 2)).reshape(x.shape[0], -1)
    h = _fc1(h, fc1_w, fc1_b)
    h = h @ fc2_w + fc2_b
    return jax.nn.log_softmax(h, axis=1)
